# baseline (speedup 1.0000x reference)
import dataclasses

import jax
import jax.numpy as jnp
from jax import lax
from jax.experimental import pallas as pl
from jax.experimental.pallas import tpu as pltpu
from jax.experimental.pallas import tpu_sc as plsc

N = 320000
D = 128
L = 16
NW = 32
ROWS_PER_W = N // NW
BLK = 400
NBLK = ROWS_PER_W // BLK
NGRP = BLK // L


def _compiler_params():
    cp = pltpu.CompilerParams()
    if "needs_layout_passes" in pltpu.CompilerParams.__dataclass_fields__:
        cp = dataclasses.replace(cp, needs_layout_passes=False)
    return cp


def _sc_mask_rows(score_flat, y):
    mesh = plsc.VectorSubcoreMesh(core_axis_name="core", subcore_axis_name="subcore")

    @pl.kernel(
        out_type=jax.ShapeDtypeStruct((N, D), jnp.float32),
        mesh=mesh,
        scratch_types=[
            pltpu.VMEM((BLK, D), jnp.float32),
            pltpu.VMEM((BLK, D), jnp.float32),
            pltpu.VMEM((BLK * 4,), jnp.float32),
            pltpu.VMEM((BLK * 4,), jnp.float32),
            pltpu.SemaphoreType.DMA,
            pltpu.SemaphoreType.DMA,
            pltpu.SemaphoreType.DMA,
            pltpu.SemaphoreType.DMA,
            pltpu.SemaphoreType.DMA,
            pltpu.SemaphoreType.DMA,
        ],
        compiler_params=_compiler_params(),
    )
    def sc_kernel(
        score_hbm, y_hbm, o_hbm, buf0, buf1, sb0, sb1, yi0, yi1, si0, si1, so0, so1
    ):
        wid = lax.axis_index("subcore") * 2 + lax.axis_index("core")
        base = wid * ROWS_PER_W
        sbase = wid * (ROWS_PER_W * 4)
        bufs = (buf0, buf1)
        sbufs = (sb0, sb1)
        yins = (yi0, yi1)
        sins = (si0, si1)
        souts = (so0, so1)

        iota = lax.iota(jnp.int32, L)
        iota4 = iota * 4
        zeros = jnp.zeros((L,), jnp.float32)
        fzeros = jnp.zeros((L,), jnp.float32)

        def start_in(i, b):
            cy = pltpu.make_async_copy(
                y_hbm.at[pl.ds(base + i * BLK, BLK), :], bufs[b], yins[b]
            )
            cy.start()
            cs = pltpu.make_async_copy(
                score_hbm.at[pl.ds(sbase + i * (BLK * 4), BLK * 4)],
                sbufs[b],
                sins[b],
            )
            cs.start()
            return cy, cs

        def mk_out(i, b):
            return pltpu.make_async_copy(
                bufs[b], o_hbm.at[pl.ds(base + i * BLK, BLK), :], souts[b]
            )

        in_copies = [None, None]
        out_copies = [None, None]
        in_copies[0] = start_in(0, 0)

        for i in range(NBLK):
            b = i % 2
            nb = (i + 1) % 2
            if out_copies[nb] is not None:
                out_copies[nb].wait()
                out_copies[nb] = None
            if i + 1 < NBLK:
                in_copies[nb] = start_in(i + 1, nb)
            cy, cs = in_copies[b]
            cy.wait()
            cs.wait()

            buf = bufs[b]
            sbuf = sbufs[b]

            @pl.loop(0, NGRP)
            def _(g):
                g64 = g * (L * 4)
                gbase = jnp.full((L,), g64, jnp.int32) + iota4
                c0 = plsc.load_gather(sbuf, [gbase])
                c1 = plsc.load_gather(sbuf, [gbase + 1])
                c2 = plsc.load_gather(sbuf, [gbase + 2])
                c3 = plsc.load_gather(sbuf, [gbase + 3])
                ss = c0 * c0 + c1 * c1 + c2 * c2 + c3 * c3
                keepf = jnp.where(ss >= 4.0, jnp.float32(1.0), jnp.float32(0.0))

                @pl.loop(0, L)
                def _(r):
                    sel = jnp.where(iota == r, keepf, fzeros)
                    m = jnp.max(sel)

                    @pl.when(m < 0.5)
                    def _():
                        row = g * L + r
                        for c in range(D // L):
                            buf[row, pl.ds(c * L, L)] = zeros

            cout = mk_out(i, b)
            cout.start()
            out_copies[b] = cout

        for b in range(2):
            if out_copies[b] is not None:
                out_copies[b].wait()

    return sc_kernel(score_flat, y)


def kernel(edge_index, score, y):
    del edge_index
    score_flat = score.reshape(N * 4)
    return _sc_mask_rows(score_flat, y)


# speedup vs baseline: 1.0691x; 1.0691x over previous
import dataclasses

import jax
import jax.numpy as jnp
from jax import lax
from jax.experimental import pallas as pl
from jax.experimental.pallas import tpu as pltpu
from jax.experimental.pallas import tpu_sc as plsc

N = 320000
D = 128
L = 16
NW = 32
ROWS_PER_W = N // NW
CHUNK = 128
NCH_MAX = (ROWS_PER_W + CHUNK - 1) // CHUNK
SBLK = 400
SFLT = SBLK * 4
NSB = ROWS_PER_W // SBLK
GRP_PER_SB = SBLK // L


def _compiler_params():
    cp = pltpu.CompilerParams()
    if "needs_layout_passes" in pltpu.CompilerParams.__dataclass_fields__:
        cp = dataclasses.replace(cp, needs_layout_passes=False)
    return cp


def _sc_mask_rows(score_flat, y):
    mesh = plsc.VectorSubcoreMesh(core_axis_name="core", subcore_axis_name="subcore")

    @pl.kernel(
        out_type=jax.ShapeDtypeStruct((N, D), jnp.float32),
        mesh=mesh,
        scratch_types=[
            pltpu.VMEM((NCH_MAX, CHUNK), jnp.int32),
            pltpu.VMEM((NCH_MAX, CHUNK), jnp.int32),
            pltpu.VMEM((CHUNK, D), jnp.float32),
            pltpu.VMEM((CHUNK, D), jnp.float32),
            pltpu.VMEM((CHUNK, D), jnp.float32),
            pltpu.VMEM((SFLT,), jnp.float32),
            pltpu.VMEM((SFLT,), jnp.float32),
            pltpu.SemaphoreType.DMA,
            pltpu.SemaphoreType.DMA,
            pltpu.SemaphoreType.DMA,
            pltpu.SemaphoreType.DMA,
            pltpu.SemaphoreType.DMA,
            pltpu.SemaphoreType.DMA,
            pltpu.SemaphoreType.DMA,
        ],
        compiler_params=_compiler_params(),
    )
    def sc_kernel(
        score_hbm, y_hbm, o_hbm,
        kidx, didx, gb0, gb1, zbuf, sb0, sb1,
        gs0, gs1, ss0, ss1, zsem, si0, si1,
    ):
        wid = lax.axis_index("subcore") * 2 + lax.axis_index("core")
        base = wid * ROWS_PER_W
        sfbase = base * 4
        iota = lax.iota(jnp.int32, L)
        iota4 = iota * 4
        zero_v = jnp.zeros((L,), jnp.float32)

        @pl.loop(0, CHUNK)
        def _(r):
            for c in range(D // L):
                zbuf[r, pl.ds(c * L, L)] = zero_v

        sbufs = (sb0, sb1)
        sins = (si0, si1)

        def start_sin(i, b):
            cp = pltpu.make_async_copy(
                score_hbm.at[pl.ds(sfbase + i * SFLT, SFLT)], sbufs[b], sins[b]
            )
            cp.start()
            return cp

        pending = [start_sin(0, 0), None]
        nk = jnp.zeros((L,), jnp.int32)
        nd = jnp.zeros((L,), jnp.int32)
        for i in range(NSB):
            b = i % 2
            nb = (i + 1) % 2
            if i + 1 < NSB:
                pending[nb] = start_sin(i + 1, nb)
            pending[b].wait()
            sbuf = sbufs[b]
            row0 = base + i * SBLK

            def grp_body(g, carry, sbuf=sbuf, row0=row0):
                nk, nd = carry
                gbase = jnp.full((L,), g * (L * 4), jnp.int32) + iota4
                c0 = plsc.load_gather(sbuf, [gbase])
                c1 = plsc.load_gather(sbuf, [gbase + 1])
                c2 = plsc.load_gather(sbuf, [gbase + 2])
                c3 = plsc.load_gather(sbuf, [gbase + 3])
                ss = c0 * c0 + c1 * c1 + c2 * c2 + c3 * c3
                keep = ss >= 4.0
                drop = jnp.logical_not(keep)
                rowg = jnp.full((L,), row0, jnp.int32) + g * L + iota
                kpos = nk + plsc.cumsum(keep.astype(jnp.int32)) - 1
                plsc.store_scatter(
                    kidx, [kpos >> 7, kpos & 127], rowg, mask=keep
                )
                dpos = nd + plsc.cumsum(drop.astype(jnp.int32)) - 1
                plsc.store_scatter(
                    didx, [dpos >> 7, dpos & 127], rowg, mask=drop
                )
                nk = nk + plsc.all_reduce_population_count(keep)
                nd = nd + plsc.all_reduce_population_count(drop)
                return (nk, nd)

            nk, nd = lax.fori_loop(0, GRP_PER_SB, grp_body, (nk, nd))

        nk_s = jnp.max(nk)
        nd_s = jnp.max(nd)
        nchk = lax.shift_right_logical(nk_s + (CHUNK - 1), 7)
        nchd = lax.shift_right_logical(nd_s + (CHUNK - 1), 7)
        zi = jnp.zeros((L,), jnp.int32)
        k0 = plsc.load_gather(kidx, [zi, zi])
        d0 = plsc.load_gather(didx, [zi, zi])
        kend = lax.shift_left(nchk, 7)
        dend = lax.shift_left(nchd, 7)
        for t in range(CHUNK // L):
            kposs = nk + iota + t * L
            plsc.store_scatter(
                kidx, [kposs >> 7, kposs & 127], k0, mask=kposs < kend
            )
            dposs = nd + iota + t * L
            plsc.store_scatter(
                didx, [dposs >> 7, dposs & 127], d0, mask=dposs < dend
            )

        def zfire(i, c):
            pltpu.make_async_copy(zbuf, o_hbm.at[didx.at[i]], zsem).start()
            return c

        lax.fori_loop(0, nchd, zfire, 0)

        gbufs = (gb0, gb1)
        gsems = (gs0, gs1)
        ssems = (ss0, ss1)

        def g_copy(i, b):
            return pltpu.make_async_copy(y_hbm.at[kidx.at[i]], gbufs[b], gsems[b])

        def s_copy(i, b):
            return pltpu.make_async_copy(gbufs[b], o_hbm.at[kidx.at[i]], ssems[b])

        @pl.when(nchk >= 1)
        def _():
            g_copy(0, 0).start()

        def b_body(i, c):
            @pl.when(i + 1 < nchk)
            def _():
                @pl.when(i % 2 == 0)
                def _():
                    @pl.when(i >= 1)
                    def _():
                        s_copy(i - 1, 1).wait()

                    g_copy(i + 1, 1).start()

                @pl.when(i % 2 == 1)
                def _():
                    s_copy(i - 1, 0).wait()
                    g_copy(i + 1, 0).start()

            @pl.when(i % 2 == 0)
            def _():
                g_copy(i, 0).wait()
                s_copy(i, 0).start()

            @pl.when(i % 2 == 1)
            def _():
                g_copy(i, 1).wait()
                s_copy(i, 1).start()

            return c

        lax.fori_loop(0, nchk, b_body, 0)

        @pl.when(nchk >= 2)
        def _():
            @pl.when(nchk % 2 == 0)
            def _():
                s_copy(nchk - 2, 0).wait()

            @pl.when(nchk % 2 == 1)
            def _():
                s_copy(nchk - 2, 1).wait()

        @pl.when(nchk >= 1)
        def _():
            @pl.when(nchk % 2 == 1)
            def _():
                s_copy(nchk - 1, 0).wait()

            @pl.when(nchk % 2 == 0)
            def _():
                s_copy(nchk - 1, 1).wait()

        def zdrain(i, c):
            pltpu.make_async_copy(zbuf, o_hbm.at[didx.at[0]], zsem).wait()
            return c

        lax.fori_loop(0, nchd, zdrain, 0)

    return sc_kernel(score_flat, y)


def kernel(edge_index, score, y):
    del edge_index
    score_flat = score.reshape(N * 4)
    return _sc_mask_rows(score_flat, y)
